# baseline (speedup 1.0000x reference)
import functools

import jax
import jax.numpy as jnp
from jax import lax
from jax.experimental import pallas as pl
from jax.experimental.pallas import tpu as pltpu
from jax.experimental.pallas import tpu_sc as plsc

D = 64
C = 128


@functools.partial(jax.jit, static_argnames=("n_ids",))
def _emb_lookup(flat_ids, weight, n_ids):
    info = plsc.get_sparse_core_info()
    nw = info.num_cores * info.num_subcores
    b_per_w = n_ids // nw
    n_chunks = b_per_w // C
    mesh = plsc.VectorSubcoreMesh(core_axis_name="c", subcore_axis_name="s")

    @functools.partial(
        pl.kernel,
        mesh=mesh,
        compiler_params=pltpu.CompilerParams(use_tc_tiling_on_sc=False),
        out_type=jax.ShapeDtypeStruct((n_ids, D), jnp.float32),
        scratch_types=[
            pltpu.VMEM((C,), jnp.int32),
            pltpu.VMEM((C, D), jnp.float32),
            pltpu.SemaphoreType.DMA,
        ],
    )
    def emb(ids_hbm, table_hbm, out_hbm, idx_v, rows_v, sem):
        wid = lax.axis_index("s") * info.num_cores + lax.axis_index("c")
        base = wid * b_per_w

        def body(i, carry):
            off = base + i * C
            pltpu.sync_copy(ids_hbm.at[pl.ds(off, C)], idx_v)
            pltpu.async_copy(table_hbm.at[idx_v], rows_v, sem).wait()
            pltpu.sync_copy(rows_v, out_hbm.at[pl.ds(off, C)])
            return carry

        lax.fori_loop(0, n_chunks, body, 0)

    return emb(flat_ids, weight)


def kernel(token_ids, weight):
    shape = token_ids.shape
    flat = token_ids.reshape(-1).astype(jnp.int32)
    out = _emb_lookup(flat, weight, flat.shape[0])
    return out.reshape(shape + (D,))


# speedup vs baseline: 1.1937x; 1.1937x over previous
import functools

import jax
import jax.numpy as jnp
from jax import lax
from jax.experimental import pallas as pl
from jax.experimental.pallas import tpu as pltpu
from jax.experimental.pallas import tpu_sc as plsc

D = 64
C = 128
NBUF = 4
LA = 2


@functools.partial(jax.jit, static_argnames=("n_ids",))
def _emb_lookup(flat_ids, weight, n_ids):
    info = plsc.get_sparse_core_info()
    nw = info.num_cores * info.num_subcores
    b_per_w = n_ids // nw
    n_chunks = b_per_w // C
    n_groups = n_chunks // NBUF
    mesh = plsc.VectorSubcoreMesh(core_axis_name="c", subcore_axis_name="s")

    @functools.partial(
        pl.kernel,
        mesh=mesh,
        compiler_params=pltpu.CompilerParams(use_tc_tiling_on_sc=False),
        out_type=jax.ShapeDtypeStruct((n_ids, D), jnp.float32),
        scratch_types=[
            pltpu.VMEM((n_chunks, C), jnp.int32),
            *[pltpu.VMEM((C, D), jnp.float32) for _ in range(NBUF)],
            *[pltpu.SemaphoreType.DMA for _ in range(2 * NBUF)],
        ],
    )
    def emb(ids_hbm, table_hbm, out_hbm, idx_all, *bufs_and_sems):
        bufs = bufs_and_sems[:NBUF]
        gsem = bufs_and_sems[NBUF:2 * NBUF]
        wsem = bufs_and_sems[2 * NBUF:]
        wid = lax.axis_index("s") * info.num_cores + lax.axis_index("c")
        base = wid * b_per_w

        def fire_gather(b, j):
            pltpu.async_copy(table_hbm.at[idx_all.at[j]], bufs[b], gsem[b])

        def drain_gather(b):
            pltpu.make_async_copy(table_hbm.at[idx_all.at[0]], bufs[b],
                                  gsem[b]).wait()

        def fire_write(b, j):
            pltpu.async_copy(bufs[b], out_hbm.at[pl.ds(base + j * C, C)],
                             wsem[b])

        def drain_write(b):
            pltpu.make_async_copy(bufs[b], out_hbm.at[pl.ds(base, C)],
                                  wsem[b]).wait()

        pltpu.sync_copy(ids_hbm.at[wid], idx_all)

        for j in range(LA):
            fire_gather(j % NBUF, j)

        def body(g, carry):
            for b in range(NBUF):
                j = g * NBUF + b
                bn = (b + LA) % NBUF
                if b + LA >= NBUF:
                    drain_write(bn)
                else:

                    @pl.when(g >= 1)
                    def _():
                        drain_write(bn)

                if b + LA < NBUF:
                    fire_gather(bn, j + LA)
                else:

                    @pl.when(g < n_groups - 1)
                    def _():
                        fire_gather(bn, j + LA)

                drain_gather(b)
                fire_write(b, j)
            return carry

        lax.fori_loop(0, n_groups, body, 0)

        for j in range(n_chunks - LA, n_chunks):
            drain_write(j % NBUF)

    return emb(flat_ids.reshape(nw, n_chunks, C), weight)


def kernel(token_ids, weight):
    shape = token_ids.shape
    flat = token_ids.reshape(-1).astype(jnp.int32)
    out = _emb_lookup(flat, weight, flat.shape[0])
    return out.reshape(shape + (D,))


# speedup vs baseline: 1.1940x; 1.0003x over previous
import functools

import jax
import jax.numpy as jnp
from jax import lax
from jax.experimental import pallas as pl
from jax.experimental.pallas import tpu as pltpu
from jax.experimental.pallas import tpu_sc as plsc

D = 64
C = 256
NBUF = 4
LA = 2


@functools.partial(jax.jit, static_argnames=("n_ids",))
def _emb_lookup(flat_ids, weight, n_ids):
    info = plsc.get_sparse_core_info()
    nw = info.num_cores * info.num_subcores
    b_per_w = n_ids // nw
    n_chunks = b_per_w // C
    n_groups = n_chunks // NBUF
    mesh = plsc.VectorSubcoreMesh(core_axis_name="c", subcore_axis_name="s")

    @functools.partial(
        pl.kernel,
        mesh=mesh,
        compiler_params=pltpu.CompilerParams(use_tc_tiling_on_sc=False),
        out_type=jax.ShapeDtypeStruct((n_ids, D), jnp.float32),
        scratch_types=[
            pltpu.VMEM((n_chunks, C), jnp.int32),
            *[pltpu.VMEM((C, D), jnp.float32) for _ in range(NBUF)],
            *[pltpu.SemaphoreType.DMA for _ in range(2 * NBUF)],
        ],
    )
    def emb(ids_hbm, table_hbm, out_hbm, idx_all, *bufs_and_sems):
        bufs = bufs_and_sems[:NBUF]
        gsem = bufs_and_sems[NBUF:2 * NBUF]
        wsem = bufs_and_sems[2 * NBUF:]
        wid = lax.axis_index("s") * info.num_cores + lax.axis_index("c")
        base = wid * b_per_w

        def fire_gather(b, j):
            pltpu.async_copy(table_hbm.at[idx_all.at[j]], bufs[b], gsem[b])

        def drain_gather(b):
            pltpu.make_async_copy(table_hbm.at[idx_all.at[0]], bufs[b],
                                  gsem[b]).wait()

        def fire_write(b, j):
            pltpu.async_copy(bufs[b], out_hbm.at[pl.ds(base + j * C, C)],
                             wsem[b])

        def drain_write(b):
            pltpu.make_async_copy(bufs[b], out_hbm.at[pl.ds(base, C)],
                                  wsem[b]).wait()

        pltpu.sync_copy(ids_hbm.at[wid], idx_all)

        for j in range(LA):
            fire_gather(j % NBUF, j)

        def body(g, carry):
            for b in range(NBUF):
                j = g * NBUF + b
                bn = (b + LA) % NBUF
                if b + LA >= NBUF:
                    drain_write(bn)
                else:

                    @pl.when(g >= 1)
                    def _():
                        drain_write(bn)

                if b + LA < NBUF:
                    fire_gather(bn, j + LA)
                else:

                    @pl.when(g < n_groups - 1)
                    def _():
                        fire_gather(bn, j + LA)

                drain_gather(b)
                fire_write(b, j)
            return carry

        lax.fori_loop(0, n_groups, body, 0)

        for j in range(n_chunks - LA, n_chunks):
            drain_write(j % NBUF)

    return emb(flat_ids.reshape(nw, n_chunks, C), weight)


def kernel(token_ids, weight):
    shape = token_ids.shape
    flat = token_ids.reshape(-1).astype(jnp.int32)
    out = _emb_lookup(flat, weight, flat.shape[0])
    return out.reshape(shape + (D,))
